# baseline (speedup 1.0000x reference)
import functools

import jax
import jax.numpy as jnp
from jax import lax
from jax.experimental import pallas as pl
from jax.experimental.pallas import tpu as pltpu
from jax.experimental.pallas import tpu_sc as plsc

N = 10000
R = 200
D = 200
HD = 100
HP = 112
NVH = HP // 16
E = 160000
NC, NS = 2, 16
ETT = E // NS
ECH = 2000
CK = 16
B = 1024
BT = B // (NC * NS)

_mesh = plsc.VectorSubcoreMesh(core_axis_name="c", subcore_axis_name="s")
_sc_params = pltpu.CompilerParams(use_tc_tiling_on_sc=False, needs_layout_passes=False)


@functools.partial(
    pl.kernel,
    out_type=jax.ShapeDtypeStruct((NC, N, HP), jnp.float32),
    mesh=_mesh,
    compiler_params=_sc_params,
    scratch_types=[
        pltpu.VMEM_SHARED((N, HP), jnp.float32),
        pltpu.VMEM((R, HP), jnp.int32),
        pltpu.VMEM((ECH,), jnp.int32),
        pltpu.VMEM((ECH,), jnp.int32),
        pltpu.VMEM((ECH,), jnp.int32),
        pltpu.VMEM((ECH,), jnp.float32),
        [pltpu.VMEM((CK, HP), jnp.float32)] * 4,
        [pltpu.VMEM((CK, HP), jnp.float32)] * 4,
        [pltpu.SemaphoreType.DMA] * 4,
        [pltpu.SemaphoreType.DMA] * 4,
    ],
)
def _edge_pass(feat_hbm, src_hbm, dst_hbm, rel_hbm, norm_hbm, wds_hbm,
               zero_hbm, out_hbm,
               agg_sh, wds_v, src_v, dst_v, rel_v, norm_v, hbufs, mbufs,
               gsems, ssems):
    cid = lax.axis_index("c")
    sid = lax.axis_index("s")
    ra, rb = 632, 520

    @pl.when(sid < NS - 1)
    def _():
        pltpu.sync_copy(zero_hbm, agg_sh.at[pl.ds(sid * ra, ra)])

    @pl.when(sid == NS - 1)
    def _():
        pltpu.sync_copy(zero_hbm.at[pl.ds(0, rb)], agg_sh.at[pl.ds((NS - 1) * ra, rb)])

    pltpu.sync_copy(wds_hbm.at[cid], wds_v)
    plsc.subcore_barrier()

    lane = lax.iota(jnp.int32, 16)
    zeros16 = jnp.zeros((16,), jnp.float32)
    ebase = sid * ETT

    for j in range(CK):
        for mb in mbufs:
            mb[j, pl.ds(96, 16)] = zeros16

    def compute_chunk(hb, mbuf, ssem, eoff):
        d16 = dst_v[pl.ds(eoff, CK)]
        r16 = rel_v[pl.ds(eoff, CK)]
        n16 = norm_v[pl.ds(eoff, CK)]

        @plsc.parallel_loop(0, HD // 2, unroll=5)
        def col_body(p):
            col0 = jnp.full((16,), 0, jnp.int32) + 2 * p
            col1 = col0 + 1
            h0 = plsc.load_gather(hb, [lane, col0])
            h1 = plsc.load_gather(hb, [lane, col1])
            w0p = plsc.load_gather(wds_v, [r16, col0])
            w1p = plsc.load_gather(wds_v, [r16, col1])
            wd0 = plsc.bitcast(w0p << 16, jnp.float32)
            ws0 = plsc.bitcast(w0p & jnp.int32(-65536), jnp.float32)
            wd1 = plsc.bitcast(w1p << 16, jnp.float32)
            ws1 = plsc.bitcast(w1p & jnp.int32(-65536), jnp.float32)
            m0 = (h0 * wd0 + h1 * ws0) * n16
            m1 = (h1 * wd1 + h0 * ws1) * n16
            plsc.store_scatter(mbuf, [lane, col0], m0)
            plsc.store_scatter(mbuf, [lane, col1], m1)
        pltpu.async_copy(mbuf, agg_sh.at[d16], ssem, add=True)

    def compute_pair(hA, hB, mA, mB, ssA, ssB, ea, eb):
        d16a = dst_v[pl.ds(ea, CK)]
        r16a = rel_v[pl.ds(ea, CK)]
        n16a = norm_v[pl.ds(ea, CK)]
        d16b = dst_v[pl.ds(eb, CK)]
        r16b = rel_v[pl.ds(eb, CK)]
        n16b = norm_v[pl.ds(eb, CK)]

        @plsc.parallel_loop(0, HD // 2, unroll=5)
        def col_body(p):
            col0 = jnp.full((16,), 0, jnp.int32) + 2 * p
            col1 = col0 + 1
            a0 = plsc.load_gather(hA, [lane, col0])
            a1 = plsc.load_gather(hA, [lane, col1])
            b0 = plsc.load_gather(hB, [lane, col0])
            b1 = plsc.load_gather(hB, [lane, col1])
            wa0 = plsc.load_gather(wds_v, [r16a, col0])
            wa1 = plsc.load_gather(wds_v, [r16a, col1])
            wb0 = plsc.load_gather(wds_v, [r16b, col0])
            wb1 = plsc.load_gather(wds_v, [r16b, col1])
            ma0 = (a0 * plsc.bitcast(wa0 << 16, jnp.float32)
                   + a1 * plsc.bitcast(wa0 & jnp.int32(-65536), jnp.float32)) * n16a
            ma1 = (a1 * plsc.bitcast(wa1 << 16, jnp.float32)
                   + a0 * plsc.bitcast(wa1 & jnp.int32(-65536), jnp.float32)) * n16a
            mb0 = (b0 * plsc.bitcast(wb0 << 16, jnp.float32)
                   + b1 * plsc.bitcast(wb0 & jnp.int32(-65536), jnp.float32)) * n16b
            mb1 = (b1 * plsc.bitcast(wb1 << 16, jnp.float32)
                   + b0 * plsc.bitcast(wb1 & jnp.int32(-65536), jnp.float32)) * n16b
            plsc.store_scatter(mA, [lane, col0], ma0)
            plsc.store_scatter(mA, [lane, col1], ma1)
            plsc.store_scatter(mB, [lane, col0], mb0)
            plsc.store_scatter(mB, [lane, col1], mb1)

        pltpu.async_copy(mA, agg_sh.at[d16a], ssA, add=True)
        pltpu.async_copy(mB, agg_sh.at[d16b], ssB, add=True)

    def wait_scatter(mbuf, ssem, eoff):
        d16 = dst_v[pl.ds(eoff, CK)]
        pltpu.make_async_copy(mbuf, agg_sh.at[d16], ssem).wait()

    def start_gather(eoff, hb, sem):
        s16 = src_v[pl.ds(eoff, CK)]
        pltpu.async_copy(feat_hbm.at[cid].at[s16], hb, sem)

    def wait_gather(eoff, hb, sem):
        s16 = src_v[pl.ds(eoff, CK)]
        pltpu.make_async_copy(feat_hbm.at[cid].at[s16], hb, sem).wait()

    def stage_body(gi, carry):
        goff = ebase + gi * ECH
        pltpu.sync_copy(src_hbm.at[pl.ds(goff, ECH)], src_v)
        pltpu.sync_copy(dst_hbm.at[pl.ds(goff, ECH)], dst_v)
        pltpu.sync_copy(rel_hbm.at[pl.ds(goff, ECH)], rel_v)
        pltpu.sync_copy(norm_hbm.at[pl.ds(goff, ECH)], norm_v)

        start_gather(0, hbufs[0], gsems[0])
        start_gather(CK, hbufs[1], gsems[1])

        def quad_body(kk, c2):
            e0 = (4 * kk) * CK

            @pl.when(kk > 0)
            def _():
                wait_scatter(mbufs[0], ssems[0], e0 - 4 * CK)
                wait_scatter(mbufs[1], ssems[1], e0 - 3 * CK)

            wait_gather(e0, hbufs[0], gsems[0])
            wait_gather(e0 + CK, hbufs[1], gsems[1])
            start_gather(e0 + 2 * CK, hbufs[2], gsems[2])
            start_gather(e0 + 3 * CK, hbufs[3], gsems[3])
            compute_pair(hbufs[0], hbufs[1], mbufs[0], mbufs[1],
                         ssems[0], ssems[1], e0, e0 + CK)

            @pl.when(kk > 0)
            def _():
                wait_scatter(mbufs[2], ssems[2], e0 - 2 * CK)
                wait_scatter(mbufs[3], ssems[3], e0 - CK)

            wait_gather(e0 + 2 * CK, hbufs[2], gsems[2])
            wait_gather(e0 + 3 * CK, hbufs[3], gsems[3])
            start_gather(e0 + 4 * CK, hbufs[0], gsems[0])

            @pl.when(kk < (ECH // CK - 1) // 4 - 1)
            def _():
                start_gather(e0 + 5 * CK, hbufs[1], gsems[1])

            compute_pair(hbufs[2], hbufs[3], mbufs[2], mbufs[3],
                         ssems[2], ssems[3], e0 + 2 * CK, e0 + 3 * CK)
            return c2

        nquads = (ECH // CK - 1) // 4
        lax.fori_loop(0, nquads, quad_body, 0)
        elast = (ECH // CK - 1) * CK
        wait_scatter(mbufs[0], ssems[0], elast - 4 * CK)
        wait_gather(elast, hbufs[0], gsems[0])
        compute_chunk(hbufs[0], mbufs[0], ssems[0], elast)
        wait_scatter(mbufs[0], ssems[0], elast)
        wait_scatter(mbufs[1], ssems[1], elast - 3 * CK)
        wait_scatter(mbufs[2], ssems[2], elast - 2 * CK)
        wait_scatter(mbufs[3], ssems[3], elast - CK)
        return carry

    lax.fori_loop(0, ETT // ECH, stage_body, 0)
    plsc.subcore_barrier()

    @pl.when(sid < NS - 1)
    def _():
        pltpu.sync_copy(agg_sh.at[pl.ds(sid * ra, ra)],
                        out_hbm.at[cid, pl.ds(sid * ra, ra)])

    @pl.when(sid == NS - 1)
    def _():
        pltpu.sync_copy(agg_sh.at[pl.ds((NS - 1) * ra, rb)],
                        out_hbm.at[cid, pl.ds((NS - 1) * ra, rb)])


def _combine_body(apply_act, agg_ref, feat_ref, w_ref, bias_ref, out_ref):
    x = jnp.concatenate([feat_ref[0], feat_ref[1]], axis=1)
    y = jnp.dot(x, w_ref[...], preferred_element_type=jnp.float32,
                precision=lax.Precision.HIGHEST)
    y = y + bias_ref[...]
    y = y + jnp.concatenate([agg_ref[0], agg_ref[1]], axis=1)
    if apply_act:
        y = jnp.maximum(y, 0.0)
    out_ref[0] = y[:, :HP]
    out_ref[1] = y[:, HP:]


def _combine(agg, feat, loop_w, bias, apply_act):
    rb = 1000
    grid = (N // rb,)
    return pl.pallas_call(
        functools.partial(_combine_body, apply_act),
        grid=grid,
        in_specs=[
            pl.BlockSpec((NC, rb, HP), lambda i: (0, i, 0)),
            pl.BlockSpec((NC, rb, HP), lambda i: (0, i, 0)),
            pl.BlockSpec((2 * HP, 2 * HP), lambda i: (0, 0)),
            pl.BlockSpec((1, 2 * HP), lambda i: (0, 0)),
        ],
        out_specs=pl.BlockSpec((NC, rb, HP), lambda i: (0, i, 0)),
        out_shape=jax.ShapeDtypeStruct((NC, N, HP), jnp.float32),
    )(agg, feat, loop_w, bias)


@functools.partial(
    pl.kernel,
    out_type=jax.ShapeDtypeStruct((B,), jnp.float32),
    mesh=_mesh,
    compiler_params=_sc_params,
    scratch_types=[
        pltpu.VMEM((BT,), jnp.int32),
        pltpu.VMEM((BT,), jnp.int32),
        pltpu.VMEM((BT,), jnp.int32),
        pltpu.VMEM((16, HP), jnp.float32),
        pltpu.VMEM((16, HP), jnp.float32),
        pltpu.VMEM((16, HP), jnp.float32),
        pltpu.VMEM((BT,), jnp.float32),
    ],
)
def _distmult(emb_hbm, relemb_hbm, h_hbm, r_hbm, t_hbm, out_hbm,
              hi_v, ri_v, ti_v, hb, rbuf, tb, acc_v):
    cid = lax.axis_index("c")
    sid = lax.axis_index("s")
    wid = cid * NS + sid
    base = wid * BT
    pltpu.sync_copy(h_hbm.at[pl.ds(base, BT)], hi_v)
    pltpu.sync_copy(r_hbm.at[pl.ds(base, BT)], ri_v)
    pltpu.sync_copy(t_hbm.at[pl.ds(base, BT)], ti_v)
    lane = lax.iota(jnp.int32, 16)
    for c in range(BT // 16):
        i16 = hi_v[pl.ds(c * 16, 16)]
        r16 = ri_v[pl.ds(c * 16, 16)]
        t16 = ti_v[pl.ds(c * 16, 16)]
        acc = jnp.zeros((16,), jnp.float32)
        for half in range(2):
            pltpu.sync_copy(emb_hbm.at[half].at[i16], hb)
            pltpu.sync_copy(relemb_hbm.at[half].at[r16], rbuf)
            pltpu.sync_copy(emb_hbm.at[half].at[t16], tb)

            def dbody(d, a):
                col = jnp.full((16,), 0, jnp.int32) + d
                hd = plsc.load_gather(hb, [lane, col])
                rd = plsc.load_gather(rbuf, [lane, col])
                td = plsc.load_gather(tb, [lane, col])
                return a + hd * rd * td

            acc = lax.fori_loop(0, HD, dbody, acc)
            plsc.subcore_barrier()
        acc_v[pl.ds(c * 16, 16)] = acc
    pltpu.sync_copy(acc_v, out_hbm.at[pl.ds(base, BT)])


def _split_rows(x):
    r = x.shape[0]
    return jnp.pad(x.reshape(r, 2, HD), ((0, 0), (0, 0), (0, HP - HD))
                   ).transpose(1, 0, 2)


def _bdd_tables(w):
    wd = jnp.stack([w[:, :, 0, 0], w[:, :, 1, 1]], axis=-1).reshape(R, D)
    ws = jnp.stack([w[:, :, 1, 0], w[:, :, 0, 1]], axis=-1).reshape(R, D)
    wd_s, ws_s = _split_rows(wd), _split_rows(ws)
    wd_b = lax.bitcast_convert_type(wd_s.astype(jnp.bfloat16), jnp.uint16
                                    ).astype(jnp.uint32)
    ws_b = lax.bitcast_convert_type(ws_s.astype(jnp.bfloat16), jnp.uint16
                                    ).astype(jnp.uint32)
    return lax.bitcast_convert_type((ws_b << 16) | wd_b, jnp.int32)


def kernel(edge_index, ent, rel, norm, triples, ent_table, rel_emb,
           w0, loop_w0, bias0, w1, loop_w1, bias1):
    f32 = jnp.float32
    feat0 = _split_rows(ent_table.astype(f32))
    relemb_p = _split_rows(rel_emb.astype(f32))
    wds0 = _bdd_tables(w0.astype(f32))
    wds1 = _bdd_tables(w1.astype(f32))

    def split_w(lw):
        return jnp.pad(lw.reshape(2, HD, 2, HD),
                       ((0, 0), (0, HP - HD), (0, 0), (0, HP - HD))
                       ).reshape(2 * HP, 2 * HP)

    lw0 = split_w(loop_w0.astype(f32))
    lw1 = split_w(loop_w1.astype(f32))
    b0 = jnp.pad(bias0.astype(f32).reshape(2, HD), ((0, 0), (0, HP - HD))
                 ).reshape(1, 2 * HP)
    b1 = jnp.pad(bias1.astype(f32).reshape(2, HD), ((0, 0), (0, HP - HD))
                 ).reshape(1, 2 * HP)

    srcp = edge_index[0].astype(jnp.int32)
    dstp = edge_index[1].astype(jnp.int32)
    relp = rel.astype(jnp.int32)
    normp = norm.reshape(E).astype(f32)
    zrows = jnp.zeros((632, HP), f32)

    agg0 = _edge_pass(feat0, srcp, dstp, relp, normp, wds0, zrows)
    emb1 = _combine(agg0, feat0, lw0, b0, True)
    agg1 = _edge_pass(emb1, srcp, dstp, relp, normp, wds1, zrows)
    emb2 = _combine(agg1, emb1, lw1, b1, False)

    score = _distmult(emb2, relemb_p,
                      triples[:, 0].astype(jnp.int32),
                      triples[:, 1].astype(jnp.int32),
                      triples[:, 2].astype(jnp.int32))
    return score.reshape(B, 1)


# speedup vs baseline: 1.0210x; 1.0210x over previous
import functools

import jax
import jax.numpy as jnp
from jax import lax
from jax.experimental import pallas as pl
from jax.experimental.pallas import tpu as pltpu
from jax.experimental.pallas import tpu_sc as plsc

N = 10000
R = 200
D = 200
HD = 100
HP = 112
NVH = HP // 16
E = 160000
NC, NS = 2, 16
ETT = E // NS
ECH = 2000
CK = 16
B = 1024
BT = B // (NC * NS)

_mesh = plsc.VectorSubcoreMesh(core_axis_name="c", subcore_axis_name="s")
_sc_params = pltpu.CompilerParams(use_tc_tiling_on_sc=False, needs_layout_passes=False)


@functools.partial(
    pl.kernel,
    out_type=jax.ShapeDtypeStruct((NC, N, HP), jnp.float32),
    mesh=_mesh,
    compiler_params=_sc_params,
    scratch_types=[
        pltpu.VMEM_SHARED((N, HP), jnp.float32),
        pltpu.VMEM((R, HP), jnp.int32),
        pltpu.VMEM((ECH,), jnp.int32),
        pltpu.VMEM((ECH,), jnp.int32),
        pltpu.VMEM((ECH,), jnp.int32),
        pltpu.VMEM((ECH,), jnp.float32),
        [pltpu.VMEM((CK, HP), jnp.float32)] * 4,
        [pltpu.VMEM((CK, HP), jnp.float32)] * 4,
        [pltpu.SemaphoreType.DMA] * 4,
        [pltpu.SemaphoreType.DMA] * 4,
    ],
)
def _edge_pass(feat_hbm, src_hbm, dst_hbm, rel_hbm, norm_hbm, wds_hbm,
               zero_hbm, out_hbm,
               agg_sh, wds_v, src_v, dst_v, rel_v, norm_v, hbufs, mbufs,
               gsems, ssems):
    cid = lax.axis_index("c")
    sid = lax.axis_index("s")
    ra, rb = 632, 520

    @pl.when(sid < NS - 1)
    def _():
        pltpu.sync_copy(zero_hbm, agg_sh.at[pl.ds(sid * ra, ra)])

    @pl.when(sid == NS - 1)
    def _():
        pltpu.sync_copy(zero_hbm.at[pl.ds(0, rb)], agg_sh.at[pl.ds((NS - 1) * ra, rb)])

    pltpu.sync_copy(wds_hbm.at[cid], wds_v)
    plsc.subcore_barrier()

    lane = lax.iota(jnp.int32, 16)
    zeros16 = jnp.zeros((16,), jnp.float32)
    ebase = sid * ETT

    for j in range(CK):
        for mb in mbufs:
            mb[j, pl.ds(96, 16)] = zeros16

    def compute_chunk(hb, mbuf, ssem, eoff):
        d16 = dst_v[pl.ds(eoff, CK)]
        r16 = rel_v[pl.ds(eoff, CK)]
        n16 = norm_v[pl.ds(eoff, CK)]

        @plsc.parallel_loop(0, HD // 2, unroll=5)
        def col_body(p):
            col0 = jnp.full((16,), 0, jnp.int32) + 2 * p
            col1 = col0 + 1
            h0 = plsc.load_gather(hb, [lane, col0])
            h1 = plsc.load_gather(hb, [lane, col1])
            w0p = plsc.load_gather(wds_v, [r16, col0])
            w1p = plsc.load_gather(wds_v, [r16, col1])
            wd0 = plsc.bitcast(w0p << 16, jnp.float32)
            ws0 = plsc.bitcast(w0p & jnp.int32(-65536), jnp.float32)
            wd1 = plsc.bitcast(w1p << 16, jnp.float32)
            ws1 = plsc.bitcast(w1p & jnp.int32(-65536), jnp.float32)
            m0 = (h0 * wd0 + h1 * ws0) * n16
            m1 = (h1 * wd1 + h0 * ws1) * n16
            plsc.store_scatter(mbuf, [lane, col0], m0)
            plsc.store_scatter(mbuf, [lane, col1], m1)
        pltpu.async_copy(mbuf, agg_sh.at[d16], ssem, add=True)

    def compute_pair(hA, hB, mA, mB, ssA, ssB, ea, eb):
        d16a = dst_v[pl.ds(ea, CK)]
        r16a = rel_v[pl.ds(ea, CK)]
        n16a = norm_v[pl.ds(ea, CK)]
        d16b = dst_v[pl.ds(eb, CK)]
        r16b = rel_v[pl.ds(eb, CK)]
        n16b = norm_v[pl.ds(eb, CK)]

        @plsc.parallel_loop(0, HD // 2, unroll=2)
        def col_body(p):
            col0 = jnp.full((16,), 0, jnp.int32) + 2 * p
            col1 = col0 + 1
            a0 = plsc.load_gather(hA, [lane, col0])
            a1 = plsc.load_gather(hA, [lane, col1])
            b0 = plsc.load_gather(hB, [lane, col0])
            b1 = plsc.load_gather(hB, [lane, col1])
            wa0 = plsc.load_gather(wds_v, [r16a, col0])
            wa1 = plsc.load_gather(wds_v, [r16a, col1])
            wb0 = plsc.load_gather(wds_v, [r16b, col0])
            wb1 = plsc.load_gather(wds_v, [r16b, col1])
            ma0 = (a0 * plsc.bitcast(wa0 << 16, jnp.float32)
                   + a1 * plsc.bitcast(wa0 & jnp.int32(-65536), jnp.float32)) * n16a
            ma1 = (a1 * plsc.bitcast(wa1 << 16, jnp.float32)
                   + a0 * plsc.bitcast(wa1 & jnp.int32(-65536), jnp.float32)) * n16a
            mb0 = (b0 * plsc.bitcast(wb0 << 16, jnp.float32)
                   + b1 * plsc.bitcast(wb0 & jnp.int32(-65536), jnp.float32)) * n16b
            mb1 = (b1 * plsc.bitcast(wb1 << 16, jnp.float32)
                   + b0 * plsc.bitcast(wb1 & jnp.int32(-65536), jnp.float32)) * n16b
            plsc.store_scatter(mA, [lane, col0], ma0)
            plsc.store_scatter(mA, [lane, col1], ma1)
            plsc.store_scatter(mB, [lane, col0], mb0)
            plsc.store_scatter(mB, [lane, col1], mb1)

        pltpu.async_copy(mA, agg_sh.at[d16a], ssA, add=True)
        pltpu.async_copy(mB, agg_sh.at[d16b], ssB, add=True)

    def wait_scatter(mbuf, ssem, eoff):
        d16 = dst_v[pl.ds(eoff, CK)]
        pltpu.make_async_copy(mbuf, agg_sh.at[d16], ssem).wait()

    def start_gather(eoff, hb, sem):
        s16 = src_v[pl.ds(eoff, CK)]
        pltpu.async_copy(feat_hbm.at[cid].at[s16], hb, sem)

    def wait_gather(eoff, hb, sem):
        s16 = src_v[pl.ds(eoff, CK)]
        pltpu.make_async_copy(feat_hbm.at[cid].at[s16], hb, sem).wait()

    def stage_body(gi, carry):
        goff = ebase + gi * ECH
        pltpu.sync_copy(src_hbm.at[pl.ds(goff, ECH)], src_v)
        pltpu.sync_copy(dst_hbm.at[pl.ds(goff, ECH)], dst_v)
        pltpu.sync_copy(rel_hbm.at[pl.ds(goff, ECH)], rel_v)
        pltpu.sync_copy(norm_hbm.at[pl.ds(goff, ECH)], norm_v)

        start_gather(0, hbufs[0], gsems[0])
        start_gather(CK, hbufs[1], gsems[1])

        def quad_body(kk, c2):
            e0 = (4 * kk) * CK

            @pl.when(kk > 0)
            def _():
                wait_scatter(mbufs[0], ssems[0], e0 - 4 * CK)
                wait_scatter(mbufs[1], ssems[1], e0 - 3 * CK)

            wait_gather(e0, hbufs[0], gsems[0])
            wait_gather(e0 + CK, hbufs[1], gsems[1])
            start_gather(e0 + 2 * CK, hbufs[2], gsems[2])
            start_gather(e0 + 3 * CK, hbufs[3], gsems[3])
            compute_pair(hbufs[0], hbufs[1], mbufs[0], mbufs[1],
                         ssems[0], ssems[1], e0, e0 + CK)

            @pl.when(kk > 0)
            def _():
                wait_scatter(mbufs[2], ssems[2], e0 - 2 * CK)
                wait_scatter(mbufs[3], ssems[3], e0 - CK)

            wait_gather(e0 + 2 * CK, hbufs[2], gsems[2])
            wait_gather(e0 + 3 * CK, hbufs[3], gsems[3])
            start_gather(e0 + 4 * CK, hbufs[0], gsems[0])

            @pl.when(kk < (ECH // CK - 1) // 4 - 1)
            def _():
                start_gather(e0 + 5 * CK, hbufs[1], gsems[1])

            compute_pair(hbufs[2], hbufs[3], mbufs[2], mbufs[3],
                         ssems[2], ssems[3], e0 + 2 * CK, e0 + 3 * CK)
            return c2

        nquads = (ECH // CK - 1) // 4
        lax.fori_loop(0, nquads, quad_body, 0)
        elast = (ECH // CK - 1) * CK
        wait_scatter(mbufs[0], ssems[0], elast - 4 * CK)
        wait_gather(elast, hbufs[0], gsems[0])
        compute_chunk(hbufs[0], mbufs[0], ssems[0], elast)
        wait_scatter(mbufs[0], ssems[0], elast)
        wait_scatter(mbufs[1], ssems[1], elast - 3 * CK)
        wait_scatter(mbufs[2], ssems[2], elast - 2 * CK)
        wait_scatter(mbufs[3], ssems[3], elast - CK)
        return carry

    lax.fori_loop(0, ETT // ECH, stage_body, 0)
    plsc.subcore_barrier()

    @pl.when(sid < NS - 1)
    def _():
        pltpu.sync_copy(agg_sh.at[pl.ds(sid * ra, ra)],
                        out_hbm.at[cid, pl.ds(sid * ra, ra)])

    @pl.when(sid == NS - 1)
    def _():
        pltpu.sync_copy(agg_sh.at[pl.ds((NS - 1) * ra, rb)],
                        out_hbm.at[cid, pl.ds((NS - 1) * ra, rb)])


def _combine_body(apply_act, agg_ref, feat_ref, w_ref, bias_ref, out_ref):
    x = jnp.concatenate([feat_ref[0], feat_ref[1]], axis=1)
    y = jnp.dot(x, w_ref[...], preferred_element_type=jnp.float32,
                precision=lax.Precision.HIGHEST)
    y = y + bias_ref[...]
    y = y + jnp.concatenate([agg_ref[0], agg_ref[1]], axis=1)
    if apply_act:
        y = jnp.maximum(y, 0.0)
    out_ref[0] = y[:, :HP]
    out_ref[1] = y[:, HP:]


def _combine(agg, feat, loop_w, bias, apply_act):
    rb = 1000
    grid = (N // rb,)
    return pl.pallas_call(
        functools.partial(_combine_body, apply_act),
        grid=grid,
        in_specs=[
            pl.BlockSpec((NC, rb, HP), lambda i: (0, i, 0)),
            pl.BlockSpec((NC, rb, HP), lambda i: (0, i, 0)),
            pl.BlockSpec((2 * HP, 2 * HP), lambda i: (0, 0)),
            pl.BlockSpec((1, 2 * HP), lambda i: (0, 0)),
        ],
        out_specs=pl.BlockSpec((NC, rb, HP), lambda i: (0, i, 0)),
        out_shape=jax.ShapeDtypeStruct((NC, N, HP), jnp.float32),
    )(agg, feat, loop_w, bias)


@functools.partial(
    pl.kernel,
    out_type=jax.ShapeDtypeStruct((B,), jnp.float32),
    mesh=_mesh,
    compiler_params=_sc_params,
    scratch_types=[
        pltpu.VMEM((BT,), jnp.int32),
        pltpu.VMEM((BT,), jnp.int32),
        pltpu.VMEM((BT,), jnp.int32),
        pltpu.VMEM((16, HP), jnp.float32),
        pltpu.VMEM((16, HP), jnp.float32),
        pltpu.VMEM((16, HP), jnp.float32),
        pltpu.VMEM((BT,), jnp.float32),
    ],
)
def _distmult(emb_hbm, relemb_hbm, h_hbm, r_hbm, t_hbm, out_hbm,
              hi_v, ri_v, ti_v, hb, rbuf, tb, acc_v):
    cid = lax.axis_index("c")
    sid = lax.axis_index("s")
    wid = cid * NS + sid
    base = wid * BT
    pltpu.sync_copy(h_hbm.at[pl.ds(base, BT)], hi_v)
    pltpu.sync_copy(r_hbm.at[pl.ds(base, BT)], ri_v)
    pltpu.sync_copy(t_hbm.at[pl.ds(base, BT)], ti_v)
    lane = lax.iota(jnp.int32, 16)
    for c in range(BT // 16):
        i16 = hi_v[pl.ds(c * 16, 16)]
        r16 = ri_v[pl.ds(c * 16, 16)]
        t16 = ti_v[pl.ds(c * 16, 16)]
        acc = jnp.zeros((16,), jnp.float32)
        for half in range(2):
            pltpu.sync_copy(emb_hbm.at[half].at[i16], hb)
            pltpu.sync_copy(relemb_hbm.at[half].at[r16], rbuf)
            pltpu.sync_copy(emb_hbm.at[half].at[t16], tb)

            def dbody(d, a):
                col = jnp.full((16,), 0, jnp.int32) + d
                hd = plsc.load_gather(hb, [lane, col])
                rd = plsc.load_gather(rbuf, [lane, col])
                td = plsc.load_gather(tb, [lane, col])
                return a + hd * rd * td

            acc = lax.fori_loop(0, HD, dbody, acc)
            plsc.subcore_barrier()
        acc_v[pl.ds(c * 16, 16)] = acc
    pltpu.sync_copy(acc_v, out_hbm.at[pl.ds(base, BT)])


def _split_rows(x):
    r = x.shape[0]
    return jnp.pad(x.reshape(r, 2, HD), ((0, 0), (0, 0), (0, HP - HD))
                   ).transpose(1, 0, 2)


def _bdd_tables(w):
    wd = jnp.stack([w[:, :, 0, 0], w[:, :, 1, 1]], axis=-1).reshape(R, D)
    ws = jnp.stack([w[:, :, 1, 0], w[:, :, 0, 1]], axis=-1).reshape(R, D)
    wd_s, ws_s = _split_rows(wd), _split_rows(ws)
    wd_b = lax.bitcast_convert_type(wd_s.astype(jnp.bfloat16), jnp.uint16
                                    ).astype(jnp.uint32)
    ws_b = lax.bitcast_convert_type(ws_s.astype(jnp.bfloat16), jnp.uint16
                                    ).astype(jnp.uint32)
    return lax.bitcast_convert_type((ws_b << 16) | wd_b, jnp.int32)


def kernel(edge_index, ent, rel, norm, triples, ent_table, rel_emb,
           w0, loop_w0, bias0, w1, loop_w1, bias1):
    f32 = jnp.float32
    feat0 = _split_rows(ent_table.astype(f32))
    relemb_p = _split_rows(rel_emb.astype(f32))
    wds0 = _bdd_tables(w0.astype(f32))
    wds1 = _bdd_tables(w1.astype(f32))

    def split_w(lw):
        return jnp.pad(lw.reshape(2, HD, 2, HD),
                       ((0, 0), (0, HP - HD), (0, 0), (0, HP - HD))
                       ).reshape(2 * HP, 2 * HP)

    lw0 = split_w(loop_w0.astype(f32))
    lw1 = split_w(loop_w1.astype(f32))
    b0 = jnp.pad(bias0.astype(f32).reshape(2, HD), ((0, 0), (0, HP - HD))
                 ).reshape(1, 2 * HP)
    b1 = jnp.pad(bias1.astype(f32).reshape(2, HD), ((0, 0), (0, HP - HD))
                 ).reshape(1, 2 * HP)

    srcp = edge_index[0].astype(jnp.int32)
    dstp = edge_index[1].astype(jnp.int32)
    relp = rel.astype(jnp.int32)
    normp = norm.reshape(E).astype(f32)
    zrows = jnp.zeros((632, HP), f32)

    agg0 = _edge_pass(feat0, srcp, dstp, relp, normp, wds0, zrows)
    emb1 = _combine(agg0, feat0, lw0, b0, True)
    agg1 = _edge_pass(emb1, srcp, dstp, relp, normp, wds1, zrows)
    emb2 = _combine(agg1, emb1, lw1, b1, False)

    score = _distmult(emb2, relemb_p,
                      triples[:, 0].astype(jnp.int32),
                      triples[:, 1].astype(jnp.int32),
                      triples[:, 2].astype(jnp.int32))
    return score.reshape(B, 1)
